# baseline (speedup 1.0000x reference)
import functools

import jax
import jax.numpy as jnp
from jax import lax
from jax.experimental import pallas as pl
from jax.experimental.pallas import tpu as pltpu
from jax.experimental.pallas import tpu_sc as plsc

N = 100000
P = 1024
K = 128
B = 16384

_CHUNK = 128


def _make_sc_gather(num_rows, table_rows):
    info = plsc.get_sparse_core_info()
    nw = info.num_cores * info.num_subcores
    b_per_w = num_rows // nw
    n_chunks = b_per_w // _CHUNK
    mesh = plsc.VectorSubcoreMesh(core_axis_name="c", subcore_axis_name="s")

    @functools.partial(
        pl.kernel,
        mesh=mesh,
        out_type=jax.ShapeDtypeStruct((num_rows, K), jnp.float32),
        scratch_types=[
            pltpu.VMEM((n_chunks, _CHUNK), jnp.int32),
            pltpu.VMEM((b_per_w, K), jnp.float32),
            pltpu.SemaphoreType.DMA,
        ],
    )
    def gather_kernel(table_hbm, idx_hbm, out_hbm, idx_v, rows_v, sem):
        wid = lax.axis_index("s") * info.num_cores + lax.axis_index("c")
        base = wid * b_per_w
        pltpu.sync_copy(idx_hbm.at[wid], idx_v)
        copies = []
        for j in range(n_chunks):
            copies.append(
                pltpu.async_copy(
                    table_hbm.at[idx_v.at[j]],
                    rows_v.at[pl.ds(j * _CHUNK, _CHUNK)],
                    sem,
                )
            )
        for c in copies:
            c.wait()
        pltpu.sync_copy(rows_v, out_hbm.at[pl.ds(base, b_per_w)])

    def run(table, idx):
        idx3 = idx.reshape(nw, n_chunks, _CHUNK)
        return gather_kernel(table, idx3)

    return run


_N_CHUNKS_B = 2
_BM = 2048

_sc_gather_chunk = _make_sc_gather(B // _N_CHUNKS_B, N)


def _matmul_body(zg_ref, w_ref, mu_ref, out_ref):
    acc = lax.dot_general(
        zg_ref[...],
        w_ref[...],
        dimension_numbers=(((1,), (1,)), ((), ())),
        preferred_element_type=jnp.float32,
    )
    out_ref[...] = acc + mu_ref[...]


def _matmul_body_aliased(dst_ref, zg_ref, w_ref, mu_ref, out_ref):
    del dst_ref
    _matmul_body(zg_ref, w_ref, mu_ref, out_ref)


def _tc_matmul_chunk(dst, zg, W, mu2d, chunk, bm):
    rows_c = zg.shape[0]
    blocks_c = rows_c // bm
    grid = (blocks_c,)
    base = chunk * blocks_c
    common = dict(
        grid=grid,
        out_specs=pl.BlockSpec((bm, P), lambda i: (base + i, 0)),
        out_shape=jax.ShapeDtypeStruct((B, P), jnp.float32),
        compiler_params=pltpu.CompilerParams(
            dimension_semantics=("arbitrary",),
        ),
    )
    zg_spec = pl.BlockSpec((bm, K), lambda i: (i, 0))
    w_spec = pl.BlockSpec((P, K), lambda i: (0, 0))
    mu_spec = pl.BlockSpec((1, P), lambda i: (0, 0))
    if dst is None:
        return pl.pallas_call(
            _matmul_body,
            in_specs=[zg_spec, w_spec, mu_spec],
            **common,
        )(zg, W, mu2d)
    return pl.pallas_call(
        _matmul_body_aliased,
        in_specs=[
            pl.BlockSpec(memory_space=pl.ANY),
            zg_spec,
            w_spec,
            mu_spec,
        ],
        input_output_aliases={0: 0},
        **common,
    )(dst, zg, W, mu2d)


def kernel(X, indices, z, W, mu):
    idx = indices.astype(jnp.int32)
    mu2d = mu.reshape(1, P)
    rows_c = B // _N_CHUNKS_B
    if _N_CHUNKS_B == 1:
        zgs = [_sc_gather_chunk(z, idx)]
    else:
        zgs = [
            _sc_gather_chunk(
                z, lax.slice(idx, (s * rows_c,), ((s + 1) * rows_c,))
            )
            for s in range(_N_CHUNKS_B)
        ]
    out = None
    for s in range(_N_CHUNKS_B):
        out = _tc_matmul_chunk(out, zgs[s], W, mu2d, s, _BM)
    return out


# speedup vs baseline: 1.0521x; 1.0521x over previous
import functools

import jax
import jax.numpy as jnp
from jax import lax
from jax.experimental import pallas as pl
from jax.experimental.pallas import tpu as pltpu
from jax.experimental.pallas import tpu_sc as plsc

N = 100000
P = 1024
K = 128
B = 16384

_CHUNK = 128


def _make_sc_gather(num_rows, table_rows):
    info = plsc.get_sparse_core_info()
    nw = info.num_cores * info.num_subcores
    b_per_w = num_rows // nw
    n_chunks = b_per_w // _CHUNK
    mesh = plsc.VectorSubcoreMesh(core_axis_name="c", subcore_axis_name="s")

    @functools.partial(
        pl.kernel,
        mesh=mesh,
        out_type=jax.ShapeDtypeStruct((num_rows, K), jnp.float32),
        scratch_types=[
            pltpu.VMEM((n_chunks, _CHUNK), jnp.int32),
            pltpu.VMEM((b_per_w, K), jnp.float32),
        ]
        + [pltpu.SemaphoreType.DMA] * (2 * n_chunks),
    )
    def gather_kernel(table_hbm, idx_hbm, out_hbm, idx_v, rows_v, *sems):
        gsems, wsems = sems[:n_chunks], sems[n_chunks:]
        wid = lax.axis_index("s") * info.num_cores + lax.axis_index("c")
        base = wid * b_per_w
        pltpu.sync_copy(idx_hbm.at[wid], idx_v)
        gathers = [
            pltpu.async_copy(
                table_hbm.at[idx_v.at[j]],
                rows_v.at[pl.ds(j * _CHUNK, _CHUNK)],
                gsems[j],
            )
            for j in range(n_chunks)
        ]
        writes = []
        for j in range(n_chunks):
            gathers[j].wait()
            writes.append(
                pltpu.async_copy(
                    rows_v.at[pl.ds(j * _CHUNK, _CHUNK)],
                    out_hbm.at[pl.ds(base + j * _CHUNK, _CHUNK)],
                    wsems[j],
                )
            )
        for w in writes:
            w.wait()

    def run(table, idx):
        idx3 = idx.reshape(nw, n_chunks, _CHUNK)
        return gather_kernel(table, idx3)

    return run


_N_CHUNKS_B = 1
_BM = 2048

_sc_gather_chunk = _make_sc_gather(B // _N_CHUNKS_B, N)


def _matmul_body(zg_ref, w_ref, mu_ref, out_ref):
    acc = lax.dot_general(
        zg_ref[...],
        w_ref[...],
        dimension_numbers=(((1,), (1,)), ((), ())),
        preferred_element_type=jnp.float32,
    )
    out_ref[...] = acc + mu_ref[...]


def _matmul_body_aliased(dst_ref, zg_ref, w_ref, mu_ref, out_ref):
    del dst_ref
    _matmul_body(zg_ref, w_ref, mu_ref, out_ref)


def _tc_matmul_chunk(dst, zg, W, mu2d, chunk, bm):
    rows_c = zg.shape[0]
    blocks_c = rows_c // bm
    grid = (blocks_c,)
    base = chunk * blocks_c
    common = dict(
        grid=grid,
        out_specs=pl.BlockSpec((bm, P), lambda i: (base + i, 0)),
        out_shape=jax.ShapeDtypeStruct((B, P), jnp.float32),
        compiler_params=pltpu.CompilerParams(
            dimension_semantics=("arbitrary",),
        ),
    )
    zg_spec = pl.BlockSpec((bm, K), lambda i: (i, 0))
    w_spec = pl.BlockSpec((P, K), lambda i: (0, 0))
    mu_spec = pl.BlockSpec((1, P), lambda i: (0, 0))
    if dst is None:
        return pl.pallas_call(
            _matmul_body,
            in_specs=[zg_spec, w_spec, mu_spec],
            **common,
        )(zg, W, mu2d)
    return pl.pallas_call(
        _matmul_body_aliased,
        in_specs=[
            pl.BlockSpec(memory_space=pl.ANY),
            zg_spec,
            w_spec,
            mu_spec,
        ],
        input_output_aliases={0: 0},
        **common,
    )(dst, zg, W, mu2d)


def kernel(X, indices, z, W, mu):
    idx = indices.astype(jnp.int32)
    mu2d = mu.reshape(1, P)
    rows_c = B // _N_CHUNKS_B
    if _N_CHUNKS_B == 1:
        zgs = [_sc_gather_chunk(z, idx)]
    else:
        zgs = [
            _sc_gather_chunk(
                z, lax.slice(idx, (s * rows_c,), ((s + 1) * rows_c,))
            )
            for s in range(_N_CHUNKS_B)
        ]
    out = None
    for s in range(_N_CHUNKS_B):
        out = _tc_matmul_chunk(out, zgs[s], W, mu2d, s, _BM)
    return out


# speedup vs baseline: 1.0642x; 1.0115x over previous
import functools

import jax
import jax.numpy as jnp
from jax import lax
from jax.experimental import pallas as pl
from jax.experimental.pallas import tpu as pltpu
from jax.experimental.pallas import tpu_sc as plsc

N = 100000
P = 1024
K = 128
B = 16384

_CHUNK = 128


def _make_sc_gather(num_rows, table_rows):
    info = plsc.get_sparse_core_info()
    nw = info.num_cores * info.num_subcores
    b_per_w = num_rows // nw
    n_chunks = b_per_w // _CHUNK
    mesh = plsc.VectorSubcoreMesh(core_axis_name="c", subcore_axis_name="s")

    @functools.partial(
        pl.kernel,
        mesh=mesh,
        out_type=jax.ShapeDtypeStruct((num_rows, K), jnp.float32),
        scratch_types=[
            pltpu.VMEM((n_chunks, _CHUNK), jnp.int32),
            pltpu.VMEM((b_per_w, K), jnp.float32),
            pltpu.SemaphoreType.DMA,
        ],
    )
    def gather_kernel(table_hbm, idx_hbm, out_hbm, idx_v, rows_v, sem):
        wid = lax.axis_index("s") * info.num_cores + lax.axis_index("c")
        base = wid * b_per_w
        pltpu.sync_copy(idx_hbm.at[wid], idx_v)
        copies = [
            pltpu.async_copy(
                table_hbm.at[idx_v.at[j]],
                rows_v.at[pl.ds(j * _CHUNK, _CHUNK)],
                sem,
            )
            for j in range(n_chunks)
        ]
        for c in copies:
            c.wait()
        pltpu.sync_copy(rows_v, out_hbm.at[pl.ds(base, b_per_w)])

    def run(table, idx):
        idx3 = idx.reshape(nw, n_chunks, _CHUNK)
        return gather_kernel(table, idx3)

    return run


_N_CHUNKS_B = 1
_BM = 2048

_sc_gather_chunk = _make_sc_gather(B // _N_CHUNKS_B, N)


def _matmul_body(zg_ref, w_ref, mu_ref, out_ref):
    acc = lax.dot_general(
        zg_ref[...],
        w_ref[...],
        dimension_numbers=(((1,), (1,)), ((), ())),
        preferred_element_type=jnp.float32,
    )
    out_ref[...] = acc + mu_ref[...]


def _matmul_body_aliased(dst_ref, zg_ref, w_ref, mu_ref, out_ref):
    del dst_ref
    _matmul_body(zg_ref, w_ref, mu_ref, out_ref)


def _tc_matmul_chunk(dst, zg, W, mu2d, chunk, bm):
    rows_c = zg.shape[0]
    blocks_c = rows_c // bm
    grid = (blocks_c,)
    base = chunk * blocks_c
    common = dict(
        grid=grid,
        out_specs=pl.BlockSpec((bm, P), lambda i: (base + i, 0)),
        out_shape=jax.ShapeDtypeStruct((B, P), jnp.float32),
        compiler_params=pltpu.CompilerParams(
            dimension_semantics=("arbitrary",),
        ),
    )
    zg_spec = pl.BlockSpec((bm, K), lambda i: (i, 0))
    w_spec = pl.BlockSpec((P, K), lambda i: (0, 0))
    mu_spec = pl.BlockSpec((1, P), lambda i: (0, 0))
    if dst is None:
        return pl.pallas_call(
            _matmul_body,
            in_specs=[zg_spec, w_spec, mu_spec],
            **common,
        )(zg, W, mu2d)
    return pl.pallas_call(
        _matmul_body_aliased,
        in_specs=[
            pl.BlockSpec(memory_space=pl.ANY),
            zg_spec,
            w_spec,
            mu_spec,
        ],
        input_output_aliases={0: 0},
        **common,
    )(dst, zg, W, mu2d)


def kernel(X, indices, z, W, mu):
    idx = indices.astype(jnp.int32)
    mu2d = mu.reshape(1, P)
    rows_c = B // _N_CHUNKS_B
    if _N_CHUNKS_B == 1:
        zgs = [_sc_gather_chunk(z, idx)]
    else:
        zgs = [
            _sc_gather_chunk(
                z, lax.slice(idx, (s * rows_c,), ((s + 1) * rows_c,))
            )
            for s in range(_N_CHUNKS_B)
        ]
    out = None
    for s in range(_N_CHUNKS_B):
        out = _tc_matmul_chunk(out, zgs[s], W, mu2d, s, _BM)
    return out
